# baseline (speedup 1.0000x reference)
import jax
import jax.numpy as jnp
from jax import lax
from jax.experimental import pallas as pl
from jax.experimental.pallas import tpu as pltpu
from jax.experimental.pallas import tpu_sc as plsc

B = 16384
IN_DIM = 128
EMB_DIM = 128
K_FIXED = 64
HIDDEN = 4

BLK = 4096
GRID = B // BLK

NUM_SC = 2
NUM_TILES = 16
NW = NUM_SC * NUM_TILES
ROWS_PER_W = B // NW
GROUPS_PER_W = ROWS_PER_W // 16


def _canon_bits(v):
    return jnp.where(v == 0.0, 0, lax.bitcast_convert_type(v, jnp.int32))


def _tc_body(x_ref, keys_ref, keys_t_ref, w1_ref, b1_ref, w2_ref,
             b2_ref, net_ref, idx_ref, grp_ref, found_sc, sel_sc):
    x = x_ref[...]
    keys = keys_ref[...]

    h = jnp.maximum(
        jnp.dot(x, w1_ref[...], preferred_element_type=jnp.float32)
        + b1_ref[...], 0.0)
    net_ref[...] = (jnp.dot(h, w2_ref[...], preferred_element_type=jnp.float32)
                    + b2_ref[...])

    row_hash = jnp.sum(_canon_bits(x), axis=1, keepdims=True)
    key_hash = jnp.sum(_canon_bits(keys_t_ref[...]), axis=0,
                       keepdims=True)
    cand = row_hash == key_hash

    idx_ref[...] = jnp.full((BLK, 1), -1, jnp.int32)

    @pl.when(jnp.any(cand))
    def _verify():
        iota = lax.broadcasted_iota(jnp.int32, (BLK, K_FIXED), 1)
        first = jnp.min(jnp.where(cand, iota, K_FIXED), axis=1,
                        keepdims=True)
        onehot = (iota == first) & cand
        has_cand = jnp.any(onehot, axis=1, keepdims=True)
        oh_f = onehot.astype(jnp.float32)
        gk = jnp.dot(oh_f, keys, preferred_element_type=jnp.float32,
                     precision=lax.Precision.HIGHEST)
        rowok = (jnp.all(x == gk, axis=1, keepdims=True)
                 & has_cand)
        idx_ref[...] = jnp.where(rowok, first, -1)

        leftover = cand & jnp.logical_not(onehot)

        @pl.when(jnp.any(leftover & jnp.logical_not(rowok)))
        def _fallback():
            found_sc[...] = jnp.zeros((BLK, 1), jnp.float32)
            sel_sc[...] = jnp.zeros((BLK, 1), jnp.float32)
            ones_col = jnp.ones((IN_DIM, 1), jnp.float32)

            def scan_key(j, carry):
                keyj = keys_ref[pl.ds(j, 1), :]
                eq = (x == keyj).astype(jnp.float32)
                cnt = jnp.dot(eq, ones_col,
                              preferred_element_type=jnp.float32,
                              precision=lax.Precision.HIGHEST)
                is_new = jnp.where(
                    (cnt == float(IN_DIM)) & (found_sc[...] == 0.0),
                    1.0, 0.0)
                found_sc[...] = found_sc[...] + is_new
                sel_sc[...] = sel_sc[...] + is_new * j.astype(jnp.float32)
                return carry

            lax.fori_loop(0, K_FIXED, scan_key, 0)
            idx_ref[...] = jnp.where(found_sc[...] > 0.0,
                                     sel_sc[...].astype(jnp.int32), -1)

    grp_ref[...] = jnp.max(
        jnp.reshape((idx_ref[...] >= 0).astype(jnp.int32), (BLK // 16, 16)),
        axis=1, keepdims=True)


def _tc_stage(x, fixed_keys, W1, b1, W2, b2):
    full = lambda i: (0, 0)
    return pl.pallas_call(
        _tc_body,
        grid=(GRID,),
        in_specs=[
            pl.BlockSpec((BLK, IN_DIM), lambda i: (i, 0)),
            pl.BlockSpec((K_FIXED, IN_DIM), full),
            pl.BlockSpec((IN_DIM, K_FIXED), full),
            pl.BlockSpec((IN_DIM, HIDDEN), full),
            pl.BlockSpec((1, HIDDEN), full),
            pl.BlockSpec((HIDDEN, EMB_DIM), full),
            pl.BlockSpec((1, EMB_DIM), full),
        ],
        out_specs=[
            pl.BlockSpec((BLK, EMB_DIM), lambda i: (i, 0)),
            pl.BlockSpec((BLK, 1), lambda i: (i, 0)),
            pl.BlockSpec((BLK // 16, 1), lambda i: (i, 0)),
        ],
        out_shape=[
            jax.ShapeDtypeStruct((B, EMB_DIM), jnp.float32),
            jax.ShapeDtypeStruct((B, 1), jnp.int32),
            jax.ShapeDtypeStruct((B // 16, 1), jnp.int32),
        ],
        scratch_shapes=[
            pltpu.VMEM((BLK, 1), jnp.float32),
            pltpu.VMEM((BLK, 1), jnp.float32),
        ],
        compiler_params=pltpu.CompilerParams(
            dimension_semantics=("arbitrary",)),
    )(x, fixed_keys, fixed_keys.T, W1, b1.reshape(1, HIDDEN), W2,
      b2.reshape(1, EMB_DIM))


def _sc_body(net_hbm, idx_hbm, grp_hbm, vals_hbm, out_hbm, idx_v, grp_v,
             buf_v, vals_v, net_sem):
    wid = (lax.axis_index("s") * NUM_SC + lax.axis_index("c")).astype(
        jnp.int32)
    base = wid * ROWS_PER_W

    net_cp = pltpu.async_copy(net_hbm.at[pl.ds(base, ROWS_PER_W)], buf_v,
                              net_sem)
    pltpu.sync_copy(idx_hbm.at[pl.ds(base, ROWS_PER_W)], idx_v)
    pltpu.sync_copy(grp_hbm.at[pl.ds(wid * GROUPS_PER_W, GROUPS_PER_W)],
                    grp_v.at[pl.ds(0, GROUPS_PER_W)])

    tflag = jnp.max(jnp.maximum(grp_v[pl.ds(0, 16)], grp_v[pl.ds(16, 16)]))
    net_cp.wait()

    @pl.when(tflag > 0)
    def _tile_overwrite():
        pltpu.sync_copy(vals_hbm, vals_v)

        def group(g, carry):
            gflag = grp_v[pl.ds(g, 16)]

            @pl.when(gflag[0] > 0)
            def _overwrite():
                v = idx_v[pl.ds(g * 16, 16)]
                mask = v >= 0
                vc = jnp.where(mask, v, 0)
                brow = g * 16 + lax.iota(jnp.int32, 16)
                for c in range(EMB_DIM):
                    csplat = jnp.full((16,), c, jnp.int32)
                    col = plsc.load_gather(vals_v, [vc, csplat])
                    plsc.store_scatter(buf_v, [brow, csplat], col, mask=mask)

            return carry

        lax.fori_loop(0, GROUPS_PER_W, group, 0)

    pltpu.sync_copy(buf_v, out_hbm.at[pl.ds(base, ROWS_PER_W)])


def _sc_stage(net, idx, grp, fixed_values):
    mesh = plsc.VectorSubcoreMesh(core_axis_name="c", subcore_axis_name="s",
                                  num_cores=NUM_SC, num_subcores=NUM_TILES)
    return pl.kernel(
        _sc_body,
        out_type=jax.ShapeDtypeStruct((B, EMB_DIM), jnp.float32),
        mesh=mesh,
        compiler_params=pltpu.CompilerParams(needs_layout_passes=False),
        scratch_types=[
            pltpu.VMEM((ROWS_PER_W,), jnp.int32),
            pltpu.VMEM((GROUPS_PER_W + 16,), jnp.int32),
            pltpu.VMEM((ROWS_PER_W, EMB_DIM), jnp.float32),
            pltpu.VMEM((K_FIXED, EMB_DIM), jnp.float32),
            pltpu.SemaphoreType.DMA,
        ],
    )(net, idx, grp, fixed_values)


@jax.jit
def kernel(x, fixed_keys, fixed_values, W1, b1, W2, b2):
    net, idx, grp = _tc_stage(x, fixed_keys, W1, b1, W2, b2)
    return _sc_stage(net, idx.reshape(B), grp.reshape(B // 16), fixed_values)


# speedup vs baseline: 1.0091x; 1.0091x over previous
import jax
import jax.numpy as jnp
from jax import lax
from jax.experimental import pallas as pl
from jax.experimental.pallas import tpu as pltpu
from jax.experimental.pallas import tpu_sc as plsc

B = 16384
IN_DIM = 128
EMB_DIM = 128
K_FIXED = 64
HIDDEN = 4

BLK = 2048
GRID = B // BLK

NUM_SC = 2
NUM_TILES = 16
NW = NUM_SC * NUM_TILES
ROWS_PER_W = B // NW
GROUPS_PER_W = ROWS_PER_W // 16


def _canon_bits(v):
    return jnp.where(v == 0.0, 0, lax.bitcast_convert_type(v, jnp.int32))


def _tc_body(x_ref, keys_ref, keys_t_ref, w1_ref, b1_ref, w2_ref,
             b2_ref, net_ref, idx_ref, grp_ref, found_sc, sel_sc):
    x = x_ref[...]
    keys = keys_ref[...]

    h = jnp.maximum(
        jnp.dot(x, w1_ref[...], preferred_element_type=jnp.float32)
        + b1_ref[...], 0.0)
    net_ref[...] = (jnp.dot(h, w2_ref[...], preferred_element_type=jnp.float32)
                    + b2_ref[...])

    row_hash = jnp.sum(_canon_bits(x), axis=1, keepdims=True)
    key_hash = jnp.sum(_canon_bits(keys_t_ref[...]), axis=0,
                       keepdims=True)
    cand = row_hash == key_hash

    idx_ref[...] = jnp.full((BLK, 1), -1, jnp.int32)

    @pl.when(jnp.any(cand))
    def _verify():
        iota = lax.broadcasted_iota(jnp.int32, (BLK, K_FIXED), 1)
        first = jnp.min(jnp.where(cand, iota, K_FIXED), axis=1,
                        keepdims=True)
        onehot = (iota == first) & cand
        has_cand = jnp.any(onehot, axis=1, keepdims=True)
        oh_f = onehot.astype(jnp.float32)
        gk = jnp.dot(oh_f, keys, preferred_element_type=jnp.float32,
                     precision=lax.Precision.HIGHEST)
        rowok = (jnp.all(x == gk, axis=1, keepdims=True)
                 & has_cand)
        idx_ref[...] = jnp.where(rowok, first, -1)

        leftover = cand & jnp.logical_not(onehot)

        @pl.when(jnp.any(leftover & jnp.logical_not(rowok)))
        def _fallback():
            found_sc[...] = jnp.zeros((BLK, 1), jnp.float32)
            sel_sc[...] = jnp.zeros((BLK, 1), jnp.float32)
            ones_col = jnp.ones((IN_DIM, 1), jnp.float32)

            def scan_key(j, carry):
                keyj = keys_ref[pl.ds(j, 1), :]
                eq = (x == keyj).astype(jnp.float32)
                cnt = jnp.dot(eq, ones_col,
                              preferred_element_type=jnp.float32,
                              precision=lax.Precision.HIGHEST)
                is_new = jnp.where(
                    (cnt == float(IN_DIM)) & (found_sc[...] == 0.0),
                    1.0, 0.0)
                found_sc[...] = found_sc[...] + is_new
                sel_sc[...] = sel_sc[...] + is_new * j.astype(jnp.float32)
                return carry

            lax.fori_loop(0, K_FIXED, scan_key, 0)
            idx_ref[...] = jnp.where(found_sc[...] > 0.0,
                                     sel_sc[...].astype(jnp.int32), -1)

    grp_ref[...] = jnp.max(
        jnp.reshape((idx_ref[...] >= 0).astype(jnp.int32), (BLK // 16, 16)),
        axis=1, keepdims=True)


def _tc_stage(x, fixed_keys, W1, b1, W2, b2):
    full = lambda i: (0, 0)
    return pl.pallas_call(
        _tc_body,
        grid=(GRID,),
        in_specs=[
            pl.BlockSpec((BLK, IN_DIM), lambda i: (i, 0)),
            pl.BlockSpec((K_FIXED, IN_DIM), full),
            pl.BlockSpec((IN_DIM, K_FIXED), full),
            pl.BlockSpec((IN_DIM, HIDDEN), full),
            pl.BlockSpec((1, HIDDEN), full),
            pl.BlockSpec((HIDDEN, EMB_DIM), full),
            pl.BlockSpec((1, EMB_DIM), full),
        ],
        out_specs=[
            pl.BlockSpec((BLK, EMB_DIM), lambda i: (i, 0)),
            pl.BlockSpec((BLK, 1), lambda i: (i, 0)),
            pl.BlockSpec((BLK // 16, 1), lambda i: (i, 0)),
        ],
        out_shape=[
            jax.ShapeDtypeStruct((B, EMB_DIM), jnp.float32),
            jax.ShapeDtypeStruct((B, 1), jnp.int32),
            jax.ShapeDtypeStruct((B // 16, 1), jnp.int32),
        ],
        scratch_shapes=[
            pltpu.VMEM((BLK, 1), jnp.float32),
            pltpu.VMEM((BLK, 1), jnp.float32),
        ],
        compiler_params=pltpu.CompilerParams(
            dimension_semantics=("arbitrary",)),
    )(x, fixed_keys, fixed_keys.T, W1, b1.reshape(1, HIDDEN), W2,
      b2.reshape(1, EMB_DIM))


def _sc_body(net_hbm, idx_hbm, grp_hbm, vals_hbm, out_hbm, idx_v, grp_v,
             buf_v, vals_v, net_sem):
    wid = (lax.axis_index("s") * NUM_SC + lax.axis_index("c")).astype(
        jnp.int32)
    base = wid * ROWS_PER_W

    net_cp = pltpu.async_copy(net_hbm.at[pl.ds(base, ROWS_PER_W)], buf_v,
                              net_sem)
    pltpu.sync_copy(idx_hbm.at[pl.ds(base, ROWS_PER_W)], idx_v)
    pltpu.sync_copy(grp_hbm.at[pl.ds(wid * GROUPS_PER_W, GROUPS_PER_W)],
                    grp_v.at[pl.ds(0, GROUPS_PER_W)])

    tflag = jnp.max(jnp.maximum(grp_v[pl.ds(0, 16)], grp_v[pl.ds(16, 16)]))
    net_cp.wait()

    @pl.when(tflag > 0)
    def _tile_overwrite():
        pltpu.sync_copy(vals_hbm, vals_v)

        def group(g, carry):
            gflag = grp_v[pl.ds(g, 16)]

            @pl.when(gflag[0] > 0)
            def _overwrite():
                v = idx_v[pl.ds(g * 16, 16)]
                mask = v >= 0
                vc = jnp.where(mask, v, 0)
                brow = g * 16 + lax.iota(jnp.int32, 16)
                for c in range(EMB_DIM):
                    csplat = jnp.full((16,), c, jnp.int32)
                    col = plsc.load_gather(vals_v, [vc, csplat])
                    plsc.store_scatter(buf_v, [brow, csplat], col, mask=mask)

            return carry

        lax.fori_loop(0, GROUPS_PER_W, group, 0)

    pltpu.sync_copy(buf_v, out_hbm.at[pl.ds(base, ROWS_PER_W)])


def _sc_stage(net, idx, grp, fixed_values):
    mesh = plsc.VectorSubcoreMesh(core_axis_name="c", subcore_axis_name="s",
                                  num_cores=NUM_SC, num_subcores=NUM_TILES)
    return pl.kernel(
        _sc_body,
        out_type=jax.ShapeDtypeStruct((B, EMB_DIM), jnp.float32),
        mesh=mesh,
        compiler_params=pltpu.CompilerParams(needs_layout_passes=False,
                                             skip_device_barrier=True),
        scratch_types=[
            pltpu.VMEM((ROWS_PER_W,), jnp.int32),
            pltpu.VMEM((GROUPS_PER_W + 16,), jnp.int32),
            pltpu.VMEM((ROWS_PER_W, EMB_DIM), jnp.float32),
            pltpu.VMEM((K_FIXED, EMB_DIM), jnp.float32),
            pltpu.SemaphoreType.DMA,
        ],
    )(net, idx, grp, fixed_values)


@jax.jit
def kernel(x, fixed_keys, fixed_values, W1, b1, W2, b2):
    net, idx, grp = _tc_stage(x, fixed_keys, W1, b1, W2, b2)
    return _sc_stage(net, idx.reshape(B), grp.reshape(B // 16), fixed_values)


# speedup vs baseline: 1.0375x; 1.0281x over previous
import jax
import jax.numpy as jnp
from jax import lax
from jax.experimental import pallas as pl
from jax.experimental.pallas import tpu as pltpu
from jax.experimental.pallas import tpu_sc as plsc

B = 16384
IN_DIM = 128
EMB_DIM = 128
K_FIXED = 64
HIDDEN = 4

BLK = 2048
GRID = B // BLK

NUM_SC = 2
NUM_TILES = 16
NW = NUM_SC * NUM_TILES
ROWS_PER_W = B // NW
GROUPS_PER_W = ROWS_PER_W // 16


def _canon_bits(v):
    return jnp.where(v == 0.0, 0, lax.bitcast_convert_type(v, jnp.int32))


def _tc_body(x_ref, keys_ref, keys_t_ref, w1_ref, b1_ref, w2_ref,
             b2_ref, net_ref, idx_ref, grp_ref, found_sc, sel_sc):
    x = x_ref[...]
    keys = keys_ref[...]

    h = jnp.maximum(
        jnp.dot(x, w1_ref[...], preferred_element_type=jnp.float32)
        + b1_ref[...], 0.0)
    net_ref[...] = (jnp.dot(h, w2_ref[...], preferred_element_type=jnp.float32)
                    + b2_ref[...])

    row_hash = jnp.sum(_canon_bits(x), axis=1, keepdims=True)
    key_hash = jnp.sum(_canon_bits(keys_t_ref[...]), axis=0,
                       keepdims=True)
    cand = row_hash == key_hash

    idx_ref[...] = jnp.full((BLK, 1), -1, jnp.int32)

    @pl.when(jnp.any(cand))
    def _verify():
        iota = lax.broadcasted_iota(jnp.int32, (BLK, K_FIXED), 1)
        first = jnp.min(jnp.where(cand, iota, K_FIXED), axis=1,
                        keepdims=True)
        onehot = (iota == first) & cand
        has_cand = jnp.any(onehot, axis=1, keepdims=True)
        oh_f = onehot.astype(jnp.float32)
        gk = jnp.dot(oh_f, keys, preferred_element_type=jnp.float32,
                     precision=lax.Precision.HIGHEST)
        rowok = (jnp.all(x == gk, axis=1, keepdims=True)
                 & has_cand)
        idx_ref[...] = jnp.where(rowok, first, -1)

        leftover = cand & jnp.logical_not(onehot)

        @pl.when(jnp.any(leftover & jnp.logical_not(rowok)))
        def _fallback():
            found_sc[...] = jnp.zeros((BLK, 1), jnp.float32)
            sel_sc[...] = jnp.zeros((BLK, 1), jnp.float32)
            ones_col = jnp.ones((IN_DIM, 1), jnp.float32)

            def scan_key(j, carry):
                keyj = keys_ref[pl.ds(j, 1), :]
                eq = (x == keyj).astype(jnp.float32)
                cnt = jnp.dot(eq, ones_col,
                              preferred_element_type=jnp.float32,
                              precision=lax.Precision.HIGHEST)
                is_new = jnp.where(
                    (cnt == float(IN_DIM)) & (found_sc[...] == 0.0),
                    1.0, 0.0)
                found_sc[...] = found_sc[...] + is_new
                sel_sc[...] = sel_sc[...] + is_new * j.astype(jnp.float32)
                return carry

            lax.fori_loop(0, K_FIXED, scan_key, 0)
            idx_ref[...] = jnp.where(found_sc[...] > 0.0,
                                     sel_sc[...].astype(jnp.int32), -1)

    grp_ref[...] = jnp.max(
        jnp.reshape((idx_ref[...] >= 0).astype(jnp.int32), (BLK // 16, 16)),
        axis=1, keepdims=True)


def _tc_stage(x, fixed_keys, W1, b1, W2, b2):
    full = lambda i: (0, 0)
    return pl.pallas_call(
        _tc_body,
        grid=(GRID,),
        in_specs=[
            pl.BlockSpec((BLK, IN_DIM), lambda i: (i, 0)),
            pl.BlockSpec((K_FIXED, IN_DIM), full),
            pl.BlockSpec((IN_DIM, K_FIXED), full),
            pl.BlockSpec((IN_DIM, HIDDEN), full),
            pl.BlockSpec((1, HIDDEN), full),
            pl.BlockSpec((HIDDEN, EMB_DIM), full),
            pl.BlockSpec((1, EMB_DIM), full),
        ],
        out_specs=[
            pl.BlockSpec((BLK, EMB_DIM), lambda i: (i, 0)),
            pl.BlockSpec((BLK, 1), lambda i: (i, 0)),
            pl.BlockSpec((BLK // 16, 1), lambda i: (i, 0)),
        ],
        out_shape=[
            jax.ShapeDtypeStruct((B, EMB_DIM), jnp.float32),
            jax.ShapeDtypeStruct((B, 1), jnp.int32),
            jax.ShapeDtypeStruct((B // 16, 1), jnp.int32),
        ],
        scratch_shapes=[
            pltpu.VMEM((BLK, 1), jnp.float32),
            pltpu.VMEM((BLK, 1), jnp.float32),
        ],
        compiler_params=pltpu.CompilerParams(
            dimension_semantics=("arbitrary",)),
    )(x, fixed_keys, fixed_keys.T, W1, b1.reshape(1, HIDDEN), W2,
      b2.reshape(1, EMB_DIM))


NCHUNK = 4
CHUNK = ROWS_PER_W // NCHUNK
GRP_PER_CHUNK = CHUNK // 16


def _sc_body(net_hbm, idx_hbm, grp_hbm, vals_hbm, out_hbm, idx_v, grp_v,
             bufs_v, vals_v, rd_sem, wr_sem):
    wid = (lax.axis_index("s") * NUM_SC + lax.axis_index("c")).astype(
        jnp.int32)
    base = wid * ROWS_PER_W

    rds = [
        pltpu.async_copy(net_hbm.at[pl.ds(base + c * CHUNK, CHUNK)],
                         bufs_v.at[c], rd_sem)
        for c in range(NCHUNK)
    ]
    pltpu.sync_copy(idx_hbm.at[pl.ds(base, ROWS_PER_W)], idx_v)
    pltpu.sync_copy(grp_hbm.at[pl.ds(wid * GROUPS_PER_W, GROUPS_PER_W)],
                    grp_v.at[pl.ds(0, GROUPS_PER_W)])

    tflag = jnp.max(jnp.maximum(grp_v[pl.ds(0, 16)], grp_v[pl.ds(16, 16)]))

    @pl.when(tflag > 0)
    def _stage_vals():
        pltpu.sync_copy(vals_hbm, vals_v)

    wrs = []
    for c in range(NCHUNK):
        rds[c].wait()
        flags_c = grp_v[pl.ds(c * GRP_PER_CHUNK, 16)]

        @pl.when(tflag > 0)
        def _chunk_overwrite(c=c, flags_c=flags_c):
            for g in range(GRP_PER_CHUNK):
                @pl.when(flags_c[g] > 0)
                def _overwrite(c=c, g=g):
                    v = idx_v[pl.ds((c * GRP_PER_CHUNK + g) * 16, 16)]
                    mask = v >= 0
                    vc = jnp.where(mask, v, 0)
                    brow = g * 16 + lax.iota(jnp.int32, 16)

                    def col_copy(cc, carry):
                        csplat = jnp.full((16,), cc, jnp.int32)
                        col = plsc.load_gather(vals_v, [vc, csplat])
                        plsc.store_scatter(bufs_v.at[c], [brow, csplat],
                                           col, mask=mask)
                        return carry

                    lax.fori_loop(0, EMB_DIM, col_copy, 0)

        wrs.append(
            pltpu.async_copy(bufs_v.at[c],
                             out_hbm.at[pl.ds(base + c * CHUNK, CHUNK)],
                             wr_sem))
    for w in wrs:
        w.wait()


def _sc_stage(net, idx, grp, fixed_values):
    mesh = plsc.VectorSubcoreMesh(core_axis_name="c", subcore_axis_name="s",
                                  num_cores=NUM_SC, num_subcores=NUM_TILES)
    return pl.kernel(
        _sc_body,
        out_type=jax.ShapeDtypeStruct((B, EMB_DIM), jnp.float32),
        mesh=mesh,
        compiler_params=pltpu.CompilerParams(needs_layout_passes=False,
                                             skip_device_barrier=True),
        scratch_types=[
            pltpu.VMEM((ROWS_PER_W,), jnp.int32),
            pltpu.VMEM((GROUPS_PER_W + 16,), jnp.int32),
            pltpu.VMEM((NCHUNK, CHUNK, EMB_DIM), jnp.float32),
            pltpu.VMEM((K_FIXED, EMB_DIM), jnp.float32),
            pltpu.SemaphoreType.DMA,
            pltpu.SemaphoreType.DMA,
        ],
    )(net, idx, grp, fixed_values)


@jax.jit
def kernel(x, fixed_keys, fixed_values, W1, b1, W2, b2):
    net, idx, grp = _tc_stage(x, fixed_keys, W1, b1, W2, b2)
    return _sc_stage(net, idx.reshape(B), grp.reshape(B // 16), fixed_values)


# speedup vs baseline: 1.0582x; 1.0200x over previous
import jax
import jax.numpy as jnp
from jax import lax
from jax.experimental import pallas as pl
from jax.experimental.pallas import tpu as pltpu
from jax.experimental.pallas import tpu_sc as plsc

B = 16384
IN_DIM = 128
EMB_DIM = 128
K_FIXED = 64
HIDDEN = 4

BLK = 2048
GRID = B // BLK

NUM_SC = 2
NUM_TILES = 16
NW = NUM_SC * NUM_TILES
ROWS_PER_W = B // NW
GROUPS_PER_W = ROWS_PER_W // 16


def _canon_bits(v):
    return jnp.where(v == 0.0, 0, lax.bitcast_convert_type(v, jnp.int32))


def _tc_body(x_ref, keys_ref, keys_t_ref, w1_ref, b1_ref, w2_ref,
             b2_ref, net_ref, idx_ref, grp_ref, found_sc, sel_sc):
    x = x_ref[...]
    keys = keys_ref[...]

    h = jnp.maximum(
        jnp.dot(x, w1_ref[...], preferred_element_type=jnp.float32)
        + b1_ref[...], 0.0)
    net_ref[...] = (jnp.dot(h, w2_ref[...], preferred_element_type=jnp.float32)
                    + b2_ref[...])

    row_hash = jnp.sum(_canon_bits(x), axis=1, keepdims=True)
    key_hash = jnp.sum(_canon_bits(keys_t_ref[...]), axis=0,
                       keepdims=True)
    cand = row_hash == key_hash

    idx_ref[...] = jnp.full((BLK, 1), -1, jnp.int32)

    @pl.when(jnp.any(cand))
    def _verify():
        iota = lax.broadcasted_iota(jnp.int32, (BLK, K_FIXED), 1)
        first = jnp.min(jnp.where(cand, iota, K_FIXED), axis=1,
                        keepdims=True)
        onehot = (iota == first) & cand
        has_cand = jnp.any(onehot, axis=1, keepdims=True)
        oh_f = onehot.astype(jnp.float32)
        gk = jnp.dot(oh_f, keys, preferred_element_type=jnp.float32,
                     precision=lax.Precision.HIGHEST)
        rowok = (jnp.all(x == gk, axis=1, keepdims=True)
                 & has_cand)
        idx_ref[...] = jnp.where(rowok, first, -1)

        leftover = cand & jnp.logical_not(onehot)

        @pl.when(jnp.any(leftover & jnp.logical_not(rowok)))
        def _fallback():
            found_sc[...] = jnp.zeros((BLK, 1), jnp.float32)
            sel_sc[...] = jnp.zeros((BLK, 1), jnp.float32)
            ones_col = jnp.ones((IN_DIM, 1), jnp.float32)

            def scan_key(j, carry):
                keyj = keys_ref[pl.ds(j, 1), :]
                eq = (x == keyj).astype(jnp.float32)
                cnt = jnp.dot(eq, ones_col,
                              preferred_element_type=jnp.float32,
                              precision=lax.Precision.HIGHEST)
                is_new = jnp.where(
                    (cnt == float(IN_DIM)) & (found_sc[...] == 0.0),
                    1.0, 0.0)
                found_sc[...] = found_sc[...] + is_new
                sel_sc[...] = sel_sc[...] + is_new * j.astype(jnp.float32)
                return carry

            lax.fori_loop(0, K_FIXED, scan_key, 0)
            idx_ref[...] = jnp.where(found_sc[...] > 0.0,
                                     sel_sc[...].astype(jnp.int32), -1)

    grp_ref[...] = jnp.max(
        jnp.reshape((idx_ref[...] >= 0).astype(jnp.int32), (BLK // 16, 16)),
        axis=1, keepdims=True)


def _tc_stage(x, fixed_keys, W1, b1, W2, b2):
    full = lambda i: (0, 0)
    return pl.pallas_call(
        _tc_body,
        grid=(GRID,),
        in_specs=[
            pl.BlockSpec((BLK, IN_DIM), lambda i: (i, 0)),
            pl.BlockSpec((K_FIXED, IN_DIM), full),
            pl.BlockSpec((IN_DIM, K_FIXED), full),
            pl.BlockSpec((IN_DIM, HIDDEN), full),
            pl.BlockSpec((1, HIDDEN), full),
            pl.BlockSpec((HIDDEN, EMB_DIM), full),
            pl.BlockSpec((1, EMB_DIM), full),
        ],
        out_specs=[
            pl.BlockSpec((BLK, EMB_DIM), lambda i: (i, 0)),
            pl.BlockSpec((BLK, 1), lambda i: (i, 0)),
            pl.BlockSpec((BLK // 16, 1), lambda i: (i, 0)),
        ],
        out_shape=[
            jax.ShapeDtypeStruct((B, EMB_DIM), jnp.float32),
            jax.ShapeDtypeStruct((B, 1), jnp.int32),
            jax.ShapeDtypeStruct((B // 16, 1), jnp.int32),
        ],
        scratch_shapes=[
            pltpu.VMEM((BLK, 1), jnp.float32),
            pltpu.VMEM((BLK, 1), jnp.float32),
        ],
        compiler_params=pltpu.CompilerParams(
            dimension_semantics=("arbitrary",)),
    )(x, fixed_keys, fixed_keys.T, W1, b1.reshape(1, HIDDEN), W2,
      b2.reshape(1, EMB_DIM))


def _sc_body(idx_hbm, grp_hbm, vals_hbm, net_ref, idx_v, grp_v, gbuf_v,
             vals_v):
    wid = (lax.axis_index("s") * NUM_SC + lax.axis_index("c")).astype(
        jnp.int32)
    base = wid * ROWS_PER_W

    pltpu.sync_copy(grp_hbm.at[pl.ds(wid * GROUPS_PER_W, GROUPS_PER_W)],
                    grp_v.at[pl.ds(0, GROUPS_PER_W)])
    f0 = grp_v[pl.ds(0, 16)]
    f1 = grp_v[pl.ds(16, 16)]

    tflag = jnp.max(jnp.maximum(f0, f1))

    @pl.when(tflag > 0)
    def _tile_overwrite():
        pltpu.sync_copy(idx_hbm.at[pl.ds(base, ROWS_PER_W)], idx_v)
        pltpu.sync_copy(vals_hbm, vals_v)
        for g in range(GROUPS_PER_W):
            gflag = f0[g] if g < 16 else f1[g - 16]

            @pl.when(gflag > 0)
            def _overwrite(g=g):
                rows = net_ref.at[pl.ds(base + g * 16, 16)]
                pltpu.sync_copy(rows, gbuf_v)
                v = idx_v[pl.ds(g * 16, 16)]
                mask = v >= 0
                vc = jnp.where(mask, v, 0)
                brow = lax.iota(jnp.int32, 16)

                def col_copy(cc, carry):
                    csplat = jnp.full((16,), cc, jnp.int32)
                    col = plsc.load_gather(vals_v, [vc, csplat])
                    plsc.store_scatter(gbuf_v, [brow, csplat], col,
                                       mask=mask)
                    return carry

                lax.fori_loop(0, EMB_DIM, col_copy, 0)
                pltpu.sync_copy(gbuf_v, rows)


def _sc_stage(idx, grp, fixed_values, net_ref):
    mesh = plsc.VectorSubcoreMesh(core_axis_name="c", subcore_axis_name="s",
                                  num_cores=NUM_SC, num_subcores=NUM_TILES)
    return pl.kernel(
        _sc_body,
        out_type=(),
        mesh=mesh,
        compiler_params=pltpu.CompilerParams(needs_layout_passes=False),
        scratch_types=[
            pltpu.VMEM((ROWS_PER_W,), jnp.int32),
            pltpu.VMEM((GROUPS_PER_W + 16,), jnp.int32),
            pltpu.VMEM((16, EMB_DIM), jnp.float32),
            pltpu.VMEM((K_FIXED, EMB_DIM), jnp.float32),
        ],
    )(idx, grp, fixed_values, net_ref)


@jax.jit
def kernel(x, fixed_keys, fixed_values, W1, b1, W2, b2):
    net, idx, grp = _tc_stage(x, fixed_keys, W1, b1, W2, b2)
    net_ref = jax.new_ref(net)
    _sc_stage(idx.reshape(B), grp.reshape(B // 16), fixed_values, net_ref)
    return jax.freeze(net_ref)


# speedup vs baseline: 1.3087x; 1.2366x over previous
import jax
import jax.numpy as jnp
from jax import lax
from jax.experimental import pallas as pl
from jax.experimental.pallas import tpu as pltpu
from jax.experimental.pallas import tpu_sc as plsc

B = 16384
IN_DIM = 128
EMB_DIM = 128
K_FIXED = 64
HIDDEN = 4

BLK = 2048
GRID = B // BLK

NUM_SC = 2
NUM_TILES = 16
NW = NUM_SC * NUM_TILES
ROWS_PER_W = B // NW
GROUPS_PER_W = ROWS_PER_W // 16


def _canon_bits(v):
    return jnp.where(v == 0.0, 0, lax.bitcast_convert_type(v, jnp.int32))


def _tc_body(x_ref, keys_ref, keys_t_ref, w1_ref, b1_ref, w2_ref,
             b2_ref, net_ref, idx_ref, grp_ref, found_sc, sel_sc):
    x = x_ref[...]
    keys = keys_ref[...]

    h = jnp.maximum(
        jnp.dot(x, w1_ref[...], preferred_element_type=jnp.float32)
        + b1_ref[...], 0.0)
    net_ref[...] = (jnp.dot(h, w2_ref[...], preferred_element_type=jnp.float32)
                    + b2_ref[...])

    row_hash = jnp.sum(_canon_bits(x), axis=1, keepdims=True)
    key_hash = jnp.sum(_canon_bits(keys_t_ref[...]), axis=0,
                       keepdims=True)
    cand = row_hash == key_hash

    idx_ref[...] = jnp.full((BLK, 1), -1, jnp.int32)

    @pl.when(jnp.any(cand))
    def _verify():
        iota = lax.broadcasted_iota(jnp.int32, (BLK, K_FIXED), 1)
        first = jnp.min(jnp.where(cand, iota, K_FIXED), axis=1,
                        keepdims=True)
        onehot = (iota == first) & cand
        has_cand = jnp.any(onehot, axis=1, keepdims=True)
        oh_f = onehot.astype(jnp.float32)
        gk = jnp.dot(oh_f, keys, preferred_element_type=jnp.float32,
                     precision=lax.Precision.HIGHEST)
        rowok = (jnp.all(x == gk, axis=1, keepdims=True)
                 & has_cand)
        idx_ref[...] = jnp.where(rowok, first, -1)

        leftover = cand & jnp.logical_not(onehot)

        @pl.when(jnp.any(leftover & jnp.logical_not(rowok)))
        def _fallback():
            found_sc[...] = jnp.zeros((BLK, 1), jnp.float32)
            sel_sc[...] = jnp.zeros((BLK, 1), jnp.float32)
            ones_col = jnp.ones((IN_DIM, 1), jnp.float32)

            def scan_key(j, carry):
                keyj = keys_ref[pl.ds(j, 1), :]
                eq = (x == keyj).astype(jnp.float32)
                cnt = jnp.dot(eq, ones_col,
                              preferred_element_type=jnp.float32,
                              precision=lax.Precision.HIGHEST)
                is_new = jnp.where(
                    (cnt == float(IN_DIM)) & (found_sc[...] == 0.0),
                    1.0, 0.0)
                found_sc[...] = found_sc[...] + is_new
                sel_sc[...] = sel_sc[...] + is_new * j.astype(jnp.float32)
                return carry

            lax.fori_loop(0, K_FIXED, scan_key, 0)
            idx_ref[...] = jnp.where(found_sc[...] > 0.0,
                                     sel_sc[...].astype(jnp.int32), -1)

    grp_ref[...] = jnp.max(
        jnp.reshape((idx_ref[...] >= 0).astype(jnp.int32), (BLK // 16, 16)),
        axis=1, keepdims=True)


def _tc_stage(x, fixed_keys, W1, b1, W2, b2):
    full = lambda i: (0, 0)
    return pl.pallas_call(
        _tc_body,
        grid=(GRID,),
        in_specs=[
            pl.BlockSpec((BLK, IN_DIM), lambda i: (i, 0)),
            pl.BlockSpec((K_FIXED, IN_DIM), full),
            pl.BlockSpec((IN_DIM, K_FIXED), full),
            pl.BlockSpec((IN_DIM, HIDDEN), full),
            pl.BlockSpec((1, HIDDEN), full),
            pl.BlockSpec((HIDDEN, EMB_DIM), full),
            pl.BlockSpec((1, EMB_DIM), full),
        ],
        out_specs=[
            pl.BlockSpec((BLK, EMB_DIM), lambda i: (i, 0)),
            pl.BlockSpec((BLK, 1), lambda i: (i, 0)),
            pl.BlockSpec((BLK // 16, 1), lambda i: (i, 0)),
        ],
        out_shape=[
            jax.ShapeDtypeStruct((B, EMB_DIM), jnp.float32),
            jax.ShapeDtypeStruct((B, 1), jnp.int32),
            jax.ShapeDtypeStruct((B // 16, 1), jnp.int32),
        ],
        scratch_shapes=[
            pltpu.VMEM((BLK, 1), jnp.float32),
            pltpu.VMEM((BLK, 1), jnp.float32),
        ],
        compiler_params=pltpu.CompilerParams(
            dimension_semantics=("arbitrary",)),
    )(x, fixed_keys, fixed_keys.T, W1, b1.reshape(1, HIDDEN), W2,
      b2.reshape(1, EMB_DIM))


NGROUPS = B // 16
GRP_WORDS = 16 * EMB_DIM


def _sc_body(idx_hbm, grp_hbm, vals_hbm, net_ref, flags_v, idx16_v, gbuf_v,
             vals_v):
    wid = (lax.axis_index("s") * NUM_SC + lax.axis_index("c")).astype(
        jnp.int32)

    pltpu.sync_copy(grp_hbm, flags_v.at[pl.ds(0, NGROUPS)])

    def handle(k, carry):
        off = wid + k * NW
        gflag = flags_v[pl.ds(off, 16)]

        @pl.when(gflag[0] > 0)
        def _overwrite():
            gbase = off * GRP_WORDS
            pltpu.sync_copy(net_ref.at[pl.ds(gbase, GRP_WORDS)], gbuf_v)
            pltpu.sync_copy(idx_hbm.at[pl.ds(off * 16, 16)],
                            idx16_v.at[pl.ds(0, 16)])
            pltpu.sync_copy(vals_hbm, vals_v)
            for r in range(16):
                s = idx16_v[pl.ds(r, 16)][0]

                @pl.when(s >= 0)
                def _row(r=r, s=s):
                    for c in range(EMB_DIM // 16):
                        gbuf_v[pl.ds(r * EMB_DIM + c * 16, 16)] = (
                            vals_v[pl.ds(s * EMB_DIM + c * 16, 16)])

            pltpu.sync_copy(gbuf_v, net_ref.at[pl.ds(gbase, GRP_WORDS)])

        return carry

    lax.fori_loop(0, NGROUPS // NW, handle, 0)


def _sc_stage(idx, grp, vals_flat, net_ref):
    mesh = plsc.VectorSubcoreMesh(core_axis_name="c", subcore_axis_name="s",
                                  num_cores=NUM_SC, num_subcores=NUM_TILES)
    return pl.kernel(
        _sc_body,
        out_type=(),
        mesh=mesh,
        compiler_params=pltpu.CompilerParams(needs_layout_passes=False),
        scratch_types=[
            pltpu.VMEM((NGROUPS + 16,), jnp.int32),
            pltpu.VMEM((32,), jnp.int32),
            pltpu.VMEM((GRP_WORDS,), jnp.float32),
            pltpu.VMEM((K_FIXED * EMB_DIM,), jnp.float32),
        ],
    )(idx, grp, vals_flat, net_ref)


@jax.jit
def kernel(x, fixed_keys, fixed_values, W1, b1, W2, b2):
    net, idx, grp = _tc_stage(x, fixed_keys, W1, b1, W2, b2)
    net_ref = jax.new_ref(net.reshape(B * EMB_DIM))
    _sc_stage(idx.reshape(B), grp.reshape(B // 16),
              fixed_values.reshape(K_FIXED * EMB_DIM), net_ref)
    return jax.freeze(net_ref).reshape(B, EMB_DIM)
